# Initial kernel scaffold; baseline (speedup 1.0000x reference)
import functools

import jax
import jax.numpy as jnp
from jax import lax
from jax.experimental import pallas as pl
from jax.experimental.pallas import tpu as pltpu
from jax.experimental.pallas import tpu_sc as plsc

B = 4096
L = 200
D = 64
NC = 2
NS = 16
NW = NC * NS
CH = 100
NCHUNK = (B * L) // (NW * CH)
NBUF = 8
PF = 4
VPR = D // 16


def _emb_body(x_hbm, tok_hbm, pos_hbm, out_hbm, idx_v, pos_v, bufs, *sems):
  gsem = sems[:NBUF]
  wsem = sems[NBUF:]
  wid = lax.axis_index("s") * NC + lax.axis_index("c")

  pltpu.sync_copy(x_hbm.at[wid], idx_v)
  pltpu.sync_copy(pos_hbm, pos_v)

  def issue_gather(c, b):
    pltpu.async_copy(tok_hbm.at[idx_v.at[c]], bufs.at[b], gsem[b])

  def wait_gather(c, b):
    pltpu.make_async_copy(tok_hbm.at[idx_v.at[c]], bufs.at[b], gsem[b]).wait()

  def issue_write(c, b):
    pltpu.async_copy(bufs.at[b], out_hbm.at[wid, c], wsem[b])

  def wait_write(c, b):
    pltpu.make_async_copy(bufs.at[b], out_hbm.at[wid, c], wsem[b]).wait()

  def add_pos(c, b):
    off = lax.rem(c, 2) * CH
    buf = bufs.at[b]

    @plsc.parallel_loop(0, CH, unroll=2)
    def _(r):
      p = off + r
      for j in range(VPR):
        s = pl.ds(j * 16, 16)
        plsc.addupdate(buf.at[r, s], pos_v[p, s])

  def do_chunk(c, b, prefetch, drain):
    wait_gather(c, b)
    add_pos(c, b)
    issue_write(c, b)
    if prefetch:
      tgt = c + PF
      bp = (b + PF) % NBUF
      if drain:
        wait_write(tgt - NBUF, bp)
      issue_gather(tgt, bp)

  for b in range(PF):
    issue_gather(jnp.int32(b), b)

  for b in range(NBUF):
    do_chunk(jnp.int32(b), b, prefetch=True, drain=(b >= PF))

  def outer(step, _):
    for b in range(NBUF):
      c = step * NBUF + b
      do_chunk(c, b, prefetch=True, drain=True)
    return 0

  lax.fori_loop(1, NCHUNK // NBUF - 1, outer, 0)

  base = jnp.int32(NCHUNK - NBUF)
  for b in range(NBUF):
    do_chunk(base + b, b, prefetch=(b < PF), drain=True)

  for b in range(PF, NBUF):
    wait_write(base + b, b)


@jax.jit
def _emb(x2, token_table, pos_table):
  mesh = plsc.VectorSubcoreMesh(core_axis_name="c", subcore_axis_name="s")
  scratch = [
      pltpu.VMEM((NCHUNK, CH), jnp.int32),
      pltpu.VMEM((L, D), jnp.float32),
      pltpu.VMEM((NBUF, CH, D), jnp.float32),
  ] + [pltpu.SemaphoreType.DMA] * (2 * NBUF)
  f = pl.kernel(
      _emb_body,
      out_type=jax.ShapeDtypeStruct((NW, NCHUNK, CH, D), jnp.float32),
      mesh=mesh,
      scratch_types=scratch,
  )
  return f(x2, token_table, pos_table)


def kernel(x, token_table, pos_table):
  b, l = x.shape
  d = token_table.shape[1]
  assert (b, l, d) == (B, L, D)
  x2 = x.astype(jnp.int32).reshape(NW, NCHUNK, CH)
  out = _emb(x2, token_table, pos_table)
  return out.reshape(B, L, D)


# speedup vs baseline: 4.2040x; 4.2040x over previous
import functools

import jax
import jax.numpy as jnp
from jax import lax
from jax.experimental import pallas as pl
from jax.experimental.pallas import tpu as pltpu
from jax.experimental.pallas import tpu_sc as plsc

B = 4096
L = 200
D = 64
NC = 2
NS = 16
NW = NC * NS
CH = 100
NCHUNK = (B * L) // (NW * CH)
NBUF = 8
PF = 4
VPR = D // 16


def _emb_body(x_hbm, tok_hbm, pos_hbm, out_hbm, idx_v, pos_v, bufs, *sems):
  gsem = sems[:NBUF]
  wsem = sems[NBUF:]
  wid = lax.axis_index("s") * NC + lax.axis_index("c")

  pltpu.sync_copy(x_hbm.at[wid], idx_v)
  pltpu.sync_copy(pos_hbm, pos_v)

  def issue_gather(c, b):
    pltpu.async_copy(tok_hbm.at[idx_v.at[c]], bufs.at[b], gsem[b])

  def wait_gather(c, b):
    pltpu.make_async_copy(tok_hbm.at[idx_v.at[c]], bufs.at[b], gsem[b]).wait()

  def issue_write(c, b):
    pltpu.async_copy(bufs.at[b], out_hbm.at[wid, c], wsem[b])

  def wait_write(c, b):
    pltpu.make_async_copy(bufs.at[b], out_hbm.at[wid, c], wsem[b]).wait()

  def add_pos(c, b):
    off = lax.rem(c, 2) * CH
    buf = bufs.at[b]

    @plsc.parallel_loop(0, CH, unroll=2)
    def _(r):
      p = off + r
      for j in range(VPR):
        s = pl.ds(j * 16, 16)
        plsc.addupdate(buf.at[r, s], pos_v[p, s])

  def do_chunk(c, b, prefetch, drain):
    wait_gather(c, b)
    add_pos(c, b)
    issue_write(c, b)
    if prefetch:
      tgt = c + PF
      bp = (b + PF) % NBUF
      if drain:
        wait_write(tgt - NBUF, bp)
      issue_gather(tgt, bp)

  for b in range(PF):
    issue_gather(jnp.int32(b), b)

  for b in range(NBUF):
    do_chunk(jnp.int32(b), b, prefetch=True, drain=(b >= PF))

  def outer(step, _):
    for b in range(NBUF):
      c = step * NBUF + b
      do_chunk(c, b, prefetch=True, drain=True)
    return 0

  lax.fori_loop(1, NCHUNK // NBUF - 1, outer, 0)

  base = jnp.int32(NCHUNK - NBUF)
  for b in range(NBUF):
    do_chunk(base + b, b, prefetch=(b < PF), drain=True)

  for b in range(NBUF):
    wait_write(base + b, b)


@jax.jit
def _emb(x2, token_table, pos_table):
  mesh = plsc.VectorSubcoreMesh(core_axis_name="c", subcore_axis_name="s")
  scratch = [
      pltpu.VMEM((NCHUNK, CH), jnp.int32),
      pltpu.VMEM((L, D), jnp.float32),
      pltpu.VMEM((NBUF, CH, D), jnp.float32),
  ] + [pltpu.SemaphoreType.DMA] * (2 * NBUF)
  f = pl.kernel(
      _emb_body,
      out_type=jax.ShapeDtypeStruct((NW, NCHUNK, CH, D), jnp.float32),
      mesh=mesh,
      scratch_types=scratch,
      compiler_params=pltpu.CompilerParams(use_tc_tiling_on_sc=False),
  )
  return f(x2, token_table, pos_table)


def kernel(x, token_table, pos_table):
  b, l = x.shape
  d = token_table.shape[1]
  assert (b, l, d) == (B, L, D)
  x2 = x.astype(jnp.int32).reshape(NW, NCHUNK, CH)
  out = _emb(x2, token_table, pos_table)
  return out.reshape(B, L, D)
